# Initial kernel scaffold; baseline (speedup 1.0000x reference)
import jax
import jax.numpy as jnp
from jax.experimental import pallas as pl


def kernel(x, labels, centers):
    raise NotImplementedError("write your pallas kernel here")


# speedup vs baseline: 1.9014x; 1.9014x over previous
import functools

import jax
import jax.numpy as jnp
from jax import lax
from jax.experimental import pallas as pl
from jax.experimental.pallas import tpu as pltpu

_B = 4096
_D = 32
_NSUB = 3
_NC = 3000
_NCP = 3072
_MARGIN = 1.0
_BX = 512
_BC = 384
_NS1 = _B // _BX
_NS2 = _NCP // _BC
_NEG = -1e30


def _tc_body(x_ref, lab_ref, cfull_ref, crows_ref, cb_ref, pfull_ref,
             pdiag_ref, out_ref):
    g = pl.program_id(0)

    cfull = cfull_ref[...]
    ones_row = jnp.ones((1, _D), jnp.float32)
    c2_full = lax.dot_general(ones_row, cfull * cfull,
                              (((1,), (1,)), ((), ())),
                              preferred_element_type=jnp.float32)
    present = jnp.max(pfull_ref[...], axis=0, keepdims=True)
    colid = lax.broadcasted_iota(jnp.int32, (1, _NCP), 1)
    colcls = colid // _NSUB
    colmask = (colid < _NC).astype(jnp.float32)

    @pl.when(g == 0)
    def _init():
        out_ref[0] = 0.0
        out_ref[1] = 0.0
        out_ref[2] = 0.0
        out_ref[3] = 0.0

    @pl.when(g < _NS1)
    def _s1():
        xb = x_ref[...]
        cbb = cb_ref[...]
        lab = lab_ref[...]
        diff = xb - cbb
        intra = jnp.sum(diff * diff, axis=1, keepdims=True)
        x2 = jnp.sum(xb * xb, axis=1, keepdims=True)
        xc = lax.dot_general(xb, cfull, (((1,), (1,)), ((), ())),
                             preferred_element_type=jnp.float32)
        d2 = x2 + c2_full - 2.0 * xc
        lcls = lab // _NSUB
        samecls = (colcls == lcls).astype(jnp.float32)
        mask2 = 1.0 - present * samecls
        r = jnp.maximum(_MARGIN + intra - d2, 0.0) * (mask2 * colmask)
        out_ref[0] += jnp.sum(intra)
        out_ref[1] += jnp.sum(r)

    @pl.when(g >= _NS1)
    def _s2():
        s = g - _NS1
        cr = crows_ref[...]
        c2col = jnp.sum(cr * cr, axis=1, keepdims=True)
        cc = lax.dot_general(cr, cfull, (((1,), (1,)), ((), ())),
                             preferred_element_type=jnp.float32)
        d2c = c2col + c2_full - 2.0 * cc

        rowid = lax.broadcasted_iota(jnp.int32, (_BC, 1), 0) + s * _BC
        rowcls = rowid // _NSUB
        colid_d = lax.broadcasted_iota(jnp.int32, (1, _BC), 1) + s * _BC
        colcls_d = colid_d // _NSUB
        pd = jnp.max(pdiag_ref[...], axis=0, keepdims=True)
        crsq = cr * cr
        c2row_d = lax.dot_general(ones_row, crsq, (((1,), (1,)), ((), ())),
                                  preferred_element_type=jnp.float32)
        ccd = lax.dot_general(cr, cr, (((1,), (1,)), ((), ())),
                              preferred_element_type=jnp.float32)
        dd = c2col + c2row_d - 2.0 * ccd

        samecls_d = rowcls == colcls_d
        eye = rowid == colid_d
        pdb = jnp.broadcast_to(pd, (_BC, _BC))
        pcol = jnp.max(jnp.where(eye, pdb, 0.0), axis=1,
                       keepdims=True)
        colm = jnp.max(jnp.where(samecls_d & (pcol > 0.5), dd, _NEG),
                       axis=0, keepdims=True)
        colmb = jnp.broadcast_to(colm, (_BC, _BC))
        dmax = jnp.max(jnp.where(samecls_d & (pdb > 0.5), colmb, _NEG),
                       axis=1, keepdims=True)
        cnt = jnp.sum(jnp.where(samecls_d, pdb, 0.0), axis=1,
                      keepdims=True)
        care = (cnt > 1.5).astype(jnp.float32)
        w = care * pcol

        samecls_full = (colcls == rowcls).astype(jnp.float32)
        mask2c = 1.0 - present * samecls_full
        r2 = jnp.maximum(_MARGIN + dmax - d2c, 0.0) * (mask2c * colmask) * w
        out_ref[2] += jnp.sum(r2)

    del out_ref


def _s1_map(g):
    return (jnp.minimum(g, _NS1 - 1), 0)


def _s2_map(g):
    return (jnp.maximum(g - _NS1, 0), 0)


def _pdiag_map(g):
    return (0, jnp.maximum(g - _NS1, 0))


_tc_call = pl.pallas_call(
    _tc_body,
    grid=(_NS1 + _NS2,),
    in_specs=[
        pl.BlockSpec((_BX, _D), _s1_map),
        pl.BlockSpec((_BX, 1), _s1_map),
        pl.BlockSpec((_NCP, _D), lambda g: (0, 0)),
        pl.BlockSpec((_BC, _D), _s2_map),
        pl.BlockSpec((_BX, _D), _s1_map),
        pl.BlockSpec((32, _NCP), lambda g: (0, 0)),
        pl.BlockSpec((32, _BC), _pdiag_map),
    ],
    out_specs=pl.BlockSpec(memory_space=pltpu.SMEM),
    out_shape=jax.ShapeDtypeStruct((4,), jnp.float32),
    compiler_params=pltpu.CompilerParams(
        dimension_semantics=("arbitrary",)),
)


def _sc_part(labels, cpad):
    presentp = jnp.zeros((32, _NCP), jnp.float32).at[0, labels].set(1.0)
    cb = cpad[labels]
    return presentp, cb


def kernel(x, labels, centers):
    c = centers.reshape(_NC, _D)
    cpad = jnp.pad(c, ((0, _NCP - _NC), (0, 0)))
    presentp, cb = _sc_part(labels, cpad)
    lab2 = labels.reshape(_B, 1)
    sums = _tc_call(x, lab2, cpad, cpad, cb, presentp, presentp)
    intraclass = sums[0] / (_B * _D * 2.0)
    triplet = sums[1] / (2.0 * _NC * _B)
    interclass = sums[2] / (_NC * _B * 2.0)
    return (intraclass, interclass, triplet)


# speedup vs baseline: 2.1754x; 1.1441x over previous
import functools

import jax
import jax.numpy as jnp
from jax import lax
from jax.experimental import pallas as pl
from jax.experimental.pallas import tpu as pltpu
from jax.experimental.pallas import tpu_sc as plsc

_B = 4096
_D = 32
_NSUB = 3
_NC = 3000
_NCP = 3072
_MARGIN = 1.0
_BX = 512
_BC = 384
_NS1 = _B // _BX
_NS2 = _NCP // _BC
_NEG = -1e30


def _tc_body(x_ref, lab_ref, cfull_ref, crows_ref, cb_ref, pfull_ref,
             pdiag_ref, out_ref):
    g = pl.program_id(0)

    cfull = cfull_ref[...]
    ones_row = jnp.ones((1, _D), jnp.float32)
    c2_full = lax.dot_general(ones_row, cfull * cfull,
                              (((1,), (1,)), ((), ())),
                              preferred_element_type=jnp.float32)
    present = jnp.max(pfull_ref[...], axis=0, keepdims=True)
    colid = lax.broadcasted_iota(jnp.int32, (1, _NCP), 1)
    colcls = colid // _NSUB
    colmask = (colid < _NC).astype(jnp.float32)

    @pl.when(g == 0)
    def _init():
        out_ref[0] = 0.0
        out_ref[1] = 0.0
        out_ref[2] = 0.0
        out_ref[3] = 0.0

    @pl.when(g < _NS1)
    def _s1():
        xb = x_ref[...]
        cbb = cb_ref[...]
        lab = lab_ref[...]
        diff = xb - cbb
        intra = jnp.sum(diff * diff, axis=1, keepdims=True)
        x2 = jnp.sum(xb * xb, axis=1, keepdims=True)
        xc = lax.dot_general(xb, cfull, (((1,), (1,)), ((), ())),
                             preferred_element_type=jnp.float32)
        d2 = x2 + c2_full - 2.0 * xc
        lcls = lab // _NSUB
        samecls = (colcls == lcls).astype(jnp.float32)
        mask2 = 1.0 - present * samecls
        r = jnp.maximum(_MARGIN + intra - d2, 0.0) * (mask2 * colmask)
        out_ref[0] += jnp.sum(intra)
        out_ref[1] += jnp.sum(r)

    @pl.when(g >= _NS1)
    def _s2():
        s = g - _NS1
        cr = crows_ref[...]
        c2col = jnp.sum(cr * cr, axis=1, keepdims=True)
        cc = lax.dot_general(cr, cfull, (((1,), (1,)), ((), ())),
                             preferred_element_type=jnp.float32)
        d2c = c2col + c2_full - 2.0 * cc

        rowid = lax.broadcasted_iota(jnp.int32, (_BC, 1), 0) + s * _BC
        rowcls = rowid // _NSUB
        colid_d = lax.broadcasted_iota(jnp.int32, (1, _BC), 1) + s * _BC
        colcls_d = colid_d // _NSUB
        pd = jnp.max(pdiag_ref[...], axis=0, keepdims=True)
        crsq = cr * cr
        c2row_d = lax.dot_general(ones_row, crsq, (((1,), (1,)), ((), ())),
                                  preferred_element_type=jnp.float32)
        ccd = lax.dot_general(cr, cr, (((1,), (1,)), ((), ())),
                              preferred_element_type=jnp.float32)
        dd = c2col + c2row_d - 2.0 * ccd

        samecls_d = rowcls == colcls_d
        eye = rowid == colid_d
        pdb = jnp.broadcast_to(pd, (_BC, _BC))
        pcol = jnp.max(jnp.where(eye, pdb, 0.0), axis=1,
                       keepdims=True)
        colm = jnp.max(jnp.where(samecls_d & (pcol > 0.5), dd, _NEG),
                       axis=0, keepdims=True)
        colmb = jnp.broadcast_to(colm, (_BC, _BC))
        dmax = jnp.max(jnp.where(samecls_d & (pdb > 0.5), colmb, _NEG),
                       axis=1, keepdims=True)
        cnt = jnp.sum(jnp.where(samecls_d, pdb, 0.0), axis=1,
                      keepdims=True)
        care = (cnt > 1.5).astype(jnp.float32)
        w = care * pcol

        samecls_full = (colcls == rowcls).astype(jnp.float32)
        mask2c = 1.0 - present * samecls_full
        r2 = jnp.maximum(_MARGIN + dmax - d2c, 0.0) * (mask2c * colmask) * w
        out_ref[2] += jnp.sum(r2)

    del out_ref


def _s1_map(g):
    return (jnp.minimum(g, _NS1 - 1), 0)


def _s2_map(g):
    return (jnp.maximum(g - _NS1, 0), 0)


def _pdiag_map(g):
    return (0, jnp.maximum(g - _NS1, 0))


_tc_call = pl.pallas_call(
    _tc_body,
    grid=(_NS1 + _NS2,),
    in_specs=[
        pl.BlockSpec((_BX, _D), _s1_map),
        pl.BlockSpec((_BX, 1), _s1_map),
        pl.BlockSpec((_NCP, _D), lambda g: (0, 0)),
        pl.BlockSpec((_BC, _D), _s2_map),
        pl.BlockSpec((_BX, _D), _s1_map),
        pl.BlockSpec((32, _NCP), lambda g: (0, 0)),
        pl.BlockSpec((32, _BC), _pdiag_map),
    ],
    out_specs=pl.BlockSpec(memory_space=pltpu.SMEM),
    out_shape=jax.ShapeDtypeStruct((4,), jnp.float32),
    compiler_params=pltpu.CompilerParams(
        dimension_semantics=("arbitrary",)),
)


_NW = 32
_BPW = _B // _NW


@functools.partial(
    pl.kernel,
    mesh=plsc.VectorSubcoreMesh(core_axis_name="c", subcore_axis_name="s"),
    out_type=jax.ShapeDtypeStruct((_B, _D), jnp.float32),
    scratch_types=[
        pltpu.VMEM((_BPW,), jnp.int32),
        pltpu.VMEM((_BPW, _D), jnp.float32),
        pltpu.SemaphoreType.DMA,
    ],
    compiler_params=pltpu.CompilerParams(use_tc_tiling_on_sc=False),
)
def _sc_gather(labels_hbm, c_hbm, cb_hbm, idx_v, rows_v, sem):
    wid = lax.axis_index("s") * 2 + lax.axis_index("c")
    base = wid * _BPW
    pltpu.sync_copy(labels_hbm.at[pl.ds(base, _BPW)], idx_v)
    pltpu.async_copy(c_hbm.at[idx_v], rows_v, sem).wait()
    pltpu.sync_copy(rows_v, cb_hbm.at[pl.ds(base, _BPW)])


@functools.partial(
    pl.kernel,
    mesh=plsc.VectorSubcoreMesh(core_axis_name="c", subcore_axis_name="s"),
    out_type=jax.ShapeDtypeStruct((_NW, _NCP), jnp.float32),
    scratch_types=[
        pltpu.VMEM((_BPW,), jnp.int32),
        pltpu.VMEM((_NCP,), jnp.float32),
    ],
    compiler_params=pltpu.CompilerParams(needs_layout_passes=False),
)
def _sc_present(labels_hbm, present_hbm, idx_v, pbuf):
    wid = lax.axis_index("s") * 2 + lax.axis_index("c")
    base = wid * _BPW
    pltpu.sync_copy(labels_hbm.at[pl.ds(base, _BPW)], idx_v)

    def _zero(i, carry):
        pbuf[pl.ds(i * 16, 16)] = jnp.zeros((16,), jnp.float32)
        return carry

    lax.fori_loop(0, _NCP // 16, _zero, 0)
    ones16 = jnp.ones((16,), jnp.float32)
    for j in range(_BPW // 16):
        plsc.store_scatter(pbuf, [idx_v[pl.ds(j * 16, 16)]], ones16)
    pltpu.sync_copy(pbuf, present_hbm.at[wid])


def _sc_part(labels, cpad):
    return _sc_present(labels), _sc_gather(labels, cpad)


def kernel(x, labels, centers):
    c = centers.reshape(_NC, _D)
    cpad = jnp.pad(c, ((0, _NCP - _NC), (0, 0)))
    presentp, cb = _sc_part(labels, cpad)
    lab2 = labels.reshape(_B, 1)
    sums = _tc_call(x, lab2, cpad, cpad, cb, presentp, presentp)
    intraclass = sums[0] / (_B * _D * 2.0)
    triplet = sums[1] / (2.0 * _NC * _B)
    interclass = sums[2] / (_NC * _B * 2.0)
    return (intraclass, interclass, triplet)


# speedup vs baseline: 2.3195x; 1.0662x over previous
import functools

import jax
import jax.numpy as jnp
from jax import lax
from jax.experimental import pallas as pl
from jax.experimental.pallas import tpu as pltpu
from jax.experimental.pallas import tpu_sc as plsc

_B = 4096
_D = 32
_NSUB = 3
_NC = 3000
_NCP = 3072
_MARGIN = 1.0
_BX = 512
_BC = 384
_NS1 = _B // _BX
_NS2 = _NCP // _BC
_NEG = -1e30


def _tc_body(x_ref, lab_ref, cfull_ref, crows_ref, cb_ref, pfull_ref,
             pdiag_ref, out_ref):
    g = pl.program_id(0)

    cfull = cfull_ref[...]
    ones_row = jnp.ones((1, _D), jnp.float32)
    c2_full = lax.dot_general(ones_row, cfull * cfull,
                              (((1,), (1,)), ((), ())),
                              preferred_element_type=jnp.float32)
    present = jnp.max(pfull_ref[...], axis=0, keepdims=True)
    colid = lax.broadcasted_iota(jnp.int32, (1, _NCP), 1)
    colcls = colid // _NSUB
    pbool = present > 0.5
    q = jnp.where(pbool, colcls, -1)
    negc2 = jnp.where(colid < _NC, -c2_full, _NEG)
    caug = jnp.concatenate(
        [cfull, jnp.ones((_NCP, 1), jnp.float32),
         negc2.reshape(_NCP, 1)], axis=1)

    @pl.when(g == 0)
    def _init():
        out_ref[0] = 0.0
        out_ref[1] = 0.0
        out_ref[2] = 0.0
        out_ref[3] = 0.0

    @pl.when(g < _NS1)
    def _s1():
        xb = x_ref[...]
        cbb = cb_ref[...]
        lab = lab_ref[...]
        diff = xb - cbb
        intra = jnp.sum(diff * diff, axis=1, keepdims=True)
        x2 = jnp.sum(xb * xb, axis=1, keepdims=True)
        b = _MARGIN + intra - x2
        xaug = jnp.concatenate([xb + xb, b, jnp.ones((_BX, 1), jnp.float32)],
                               axis=1)
        t = lax.dot_general(xaug, caug, (((1,), (1,)), ((), ())),
                            preferred_element_type=jnp.float32)
        lcls = lab // _NSUB
        r = jnp.where(q == lcls, 0.0, jnp.maximum(t, 0.0))
        out_ref[0] += jnp.sum(intra)
        out_ref[1] += jnp.sum(r)

    @pl.when(g >= _NS1)
    def _s2():
        s = g - _NS1
        cr = crows_ref[...]
        c2col = jnp.sum(cr * cr, axis=1, keepdims=True)

        rowid = lax.broadcasted_iota(jnp.int32, (_BC, 1), 0) + s * _BC
        rowcls = rowid // _NSUB
        colid_d = lax.broadcasted_iota(jnp.int32, (1, _BC), 1) + s * _BC
        colcls_d = colid_d // _NSUB
        pd = jnp.max(pdiag_ref[...], axis=0, keepdims=True)
        crsq = cr * cr
        c2row_d = lax.dot_general(ones_row, crsq, (((1,), (1,)), ((), ())),
                                  preferred_element_type=jnp.float32)
        ccd = lax.dot_general(cr, cr, (((1,), (1,)), ((), ())),
                              preferred_element_type=jnp.float32)
        dd = c2col + c2row_d - 2.0 * ccd

        samecls_d = rowcls == colcls_d
        eye = rowid == colid_d
        pdb = jnp.broadcast_to(pd, (_BC, _BC))
        pcol = jnp.max(jnp.where(eye, pdb, 0.0), axis=1,
                       keepdims=True)
        colm = jnp.max(jnp.where(samecls_d & (pcol > 0.5), dd, _NEG),
                       axis=0, keepdims=True)
        colmb = jnp.broadcast_to(colm, (_BC, _BC))
        dmax = jnp.max(jnp.where(samecls_d & (pdb > 0.5), colmb, _NEG),
                       axis=1, keepdims=True)
        cnt = jnp.sum(jnp.where(samecls_d, pdb, 0.0), axis=1,
                      keepdims=True)
        care = (cnt > 1.5).astype(jnp.float32)
        w = care * pcol

        b2 = jnp.where(w > 0.5, _MARGIN + dmax - c2col, _NEG)
        craug = jnp.concatenate([cr + cr, b2, jnp.ones((_BC, 1), jnp.float32)],
                                axis=1)
        t2 = lax.dot_general(craug, caug, (((1,), (1,)), ((), ())),
                             preferred_element_type=jnp.float32)
        r2 = jnp.where(q == rowcls, 0.0, jnp.maximum(t2, 0.0))
        out_ref[2] += jnp.sum(r2)

    del out_ref


def _s1_map(g):
    return (jnp.minimum(g, _NS1 - 1), 0)


def _s2_map(g):
    return (jnp.maximum(g - _NS1, 0), 0)


def _pdiag_map(g):
    return (0, jnp.maximum(g - _NS1, 0))


_tc_call = pl.pallas_call(
    _tc_body,
    grid=(_NS1 + _NS2,),
    in_specs=[
        pl.BlockSpec((_BX, _D), _s1_map),
        pl.BlockSpec((_BX, 1), _s1_map),
        pl.BlockSpec((_NCP, _D), lambda g: (0, 0)),
        pl.BlockSpec((_BC, _D), _s2_map),
        pl.BlockSpec((_BX, _D), _s1_map),
        pl.BlockSpec((32, _NCP), lambda g: (0, 0)),
        pl.BlockSpec((32, _BC), _pdiag_map),
    ],
    out_specs=pl.BlockSpec(memory_space=pltpu.SMEM),
    out_shape=jax.ShapeDtypeStruct((4,), jnp.float32),
    compiler_params=pltpu.CompilerParams(
        dimension_semantics=("arbitrary",)),
)


_NW = 32
_BPW = _B // _NW


@functools.cache
def _sc_kernels():
    mesh = plsc.VectorSubcoreMesh(core_axis_name="c", subcore_axis_name="s")

    @functools.partial(
        pl.kernel,
        mesh=mesh,
        out_type=jax.ShapeDtypeStruct((_B, _D), jnp.float32),
        scratch_types=[
            pltpu.VMEM((_BPW,), jnp.int32),
            pltpu.VMEM((_BPW, _D), jnp.float32),
            pltpu.SemaphoreType.DMA,
        ],
        compiler_params=pltpu.CompilerParams(use_tc_tiling_on_sc=False),
    )
    def _sc_gather(labels_hbm, c_hbm, cb_hbm, idx_v, rows_v, sem):
        wid = lax.axis_index("s") * 2 + lax.axis_index("c")
        base = wid * _BPW
        pltpu.sync_copy(labels_hbm.at[pl.ds(base, _BPW)], idx_v)
        pltpu.async_copy(c_hbm.at[idx_v], rows_v, sem).wait()
        pltpu.sync_copy(rows_v, cb_hbm.at[pl.ds(base, _BPW)])

    @functools.partial(
        pl.kernel,
        mesh=mesh,
        out_type=jax.ShapeDtypeStruct((_NW, _NCP), jnp.float32),
        scratch_types=[
            pltpu.VMEM((_BPW,), jnp.int32),
            pltpu.VMEM((_NCP,), jnp.float32),
        ],
        compiler_params=pltpu.CompilerParams(needs_layout_passes=False),
    )
    def _sc_present(labels_hbm, present_hbm, idx_v, pbuf):
        wid = lax.axis_index("s") * 2 + lax.axis_index("c")
        base = wid * _BPW
        pltpu.sync_copy(labels_hbm.at[pl.ds(base, _BPW)], idx_v)

        def _zero(i, carry):
            pbuf[pl.ds(i * 16, 16)] = jnp.zeros((16,), jnp.float32)
            return carry

        lax.fori_loop(0, _NCP // 16, _zero, 0)
        ones16 = jnp.ones((16,), jnp.float32)
        for j in range(_BPW // 16):
            plsc.store_scatter(pbuf, [idx_v[pl.ds(j * 16, 16)]], ones16)
        pltpu.sync_copy(pbuf, present_hbm.at[wid])

    return _sc_present, _sc_gather


def _sc_part(labels, cpad):
    sc_present, sc_gather = _sc_kernels()
    return sc_present(labels), sc_gather(labels, cpad)


def kernel(x, labels, centers):
    c = centers.reshape(_NC, _D)
    cpad = jnp.pad(c, ((0, _NCP - _NC), (0, 0)))
    presentp, cb = _sc_part(labels, cpad)
    lab2 = labels.reshape(_B, 1)
    sums = _tc_call(x, lab2, cpad, cpad, cb, presentp, presentp)
    intraclass = sums[0] / (_B * _D * 2.0)
    triplet = sums[1] / (2.0 * _NC * _B)
    interclass = sums[2] / (_NC * _B * 2.0)
    return (intraclass, interclass, triplet)


# speedup vs baseline: 2.8955x; 1.2483x over previous
import functools

import jax
import jax.numpy as jnp
from jax import lax
from jax.experimental import pallas as pl
from jax.experimental.pallas import tpu as pltpu
from jax.experimental.pallas import tpu_sc as plsc

_B = 4096
_D = 32
_NSUB = 3
_NC = 3000
_NCP = 3072
_MARGIN = 1.0
_BX = 512
_BC = 384
_NS1 = _B // _BX
_NS2 = _NCP // _BC
_NEG = -1e30


def _tc_body(x_ref, lab_ref, cfull_ref, crows_ref, cb_ref, pfull_ref,
             pdiag_ref, out_ref):
    g = pl.program_id(0)

    cfull = cfull_ref[...]
    ones_row = jnp.ones((1, _D), jnp.float32)
    c2_full = lax.dot_general(ones_row, cfull * cfull,
                              (((1,), (1,)), ((), ())),
                              preferred_element_type=jnp.float32)
    present = jnp.max(pfull_ref[...], axis=0, keepdims=True)
    colid = lax.broadcasted_iota(jnp.int32, (1, _NCP), 1)
    colcls = colid // _NSUB
    pbool = present > 0.5
    q = jnp.where(pbool, colcls, -1)
    negc2 = jnp.where(colid < _NC, -c2_full, _NEG)
    caug = jnp.concatenate(
        [cfull, jnp.ones((_NCP, 1), jnp.float32),
         negc2.reshape(_NCP, 1)], axis=1)

    @pl.when(g == 0)
    def _init():
        out_ref[0] = 0.0
        out_ref[1] = 0.0
        out_ref[2] = 0.0
        out_ref[3] = 0.0

    xb = x_ref[...]
    cbb = cb_ref[...]
    lab = lab_ref[...]
    diff = xb - cbb
    intra = jnp.sum(diff * diff, axis=1, keepdims=True)
    x2 = jnp.sum(xb * xb, axis=1, keepdims=True)
    b = _MARGIN + intra - x2
    xaug = jnp.concatenate([xb + xb, b, jnp.ones((_BX, 1), jnp.float32)],
                           axis=1)
    t = lax.dot_general(xaug, caug, (((1,), (1,)), ((), ())),
                        preferred_element_type=jnp.float32)
    lcls = lab // _NSUB
    r = jnp.where(q == lcls, 0.0, jnp.maximum(t, 0.0))
    out_ref[0] += jnp.sum(intra)
    out_ref[1] += jnp.sum(r)

    cr = crows_ref[...]
    c2col = jnp.sum(cr * cr, axis=1, keepdims=True)

    rowid = lax.broadcasted_iota(jnp.int32, (_BC, 1), 0) + g * _BC
    rowcls = rowid // _NSUB
    colid_d = lax.broadcasted_iota(jnp.int32, (1, _BC), 1) + g * _BC
    colcls_d = colid_d // _NSUB
    pd = jnp.max(pdiag_ref[...], axis=0, keepdims=True)
    crsq = cr * cr
    c2row_d = lax.dot_general(ones_row, crsq, (((1,), (1,)), ((), ())),
                              preferred_element_type=jnp.float32)
    ccd = lax.dot_general(cr, cr, (((1,), (1,)), ((), ())),
                          preferred_element_type=jnp.float32)
    dd = c2col + c2row_d - 2.0 * ccd

    samecls_d = rowcls == colcls_d
    eye = rowid == colid_d
    pdb = jnp.broadcast_to(pd, (_BC, _BC))
    pcol = jnp.max(jnp.where(eye, pdb, 0.0), axis=1,
                   keepdims=True)
    colm = jnp.max(jnp.where(samecls_d & (pcol > 0.5), dd, _NEG),
                   axis=0, keepdims=True)
    colmb = jnp.broadcast_to(colm, (_BC, _BC))
    dmax = jnp.max(jnp.where(samecls_d & (pdb > 0.5), colmb, _NEG),
                   axis=1, keepdims=True)
    cnt = jnp.sum(jnp.where(samecls_d, pdb, 0.0), axis=1,
                  keepdims=True)
    care = (cnt > 1.5).astype(jnp.float32)
    w = care * pcol

    b2 = jnp.where(w > 0.5, _MARGIN + dmax - c2col, _NEG)
    craug = jnp.concatenate([cr + cr, b2, jnp.ones((_BC, 1), jnp.float32)],
                            axis=1)
    t2 = lax.dot_general(craug, caug, (((1,), (1,)), ((), ())),
                         preferred_element_type=jnp.float32)
    r2 = jnp.maximum(t2, 0.0)
    t2d = 2.0 * ccd + b2 - c2row_d
    qd = jnp.where(pd > 0.5, colcls_d, -1)
    corr = jnp.where(qd == rowcls, jnp.maximum(t2d, 0.0), 0.0)
    out_ref[2] += jnp.sum(r2) - jnp.sum(corr)

    del out_ref


def _s1_map(g):
    return (g, 0)


def _s2_map(g):
    return (g, 0)


def _pdiag_map(g):
    return (0, g)


_tc_call = pl.pallas_call(
    _tc_body,
    grid=(_NS1,),
    in_specs=[
        pl.BlockSpec((_BX, _D), _s1_map),
        pl.BlockSpec((_BX, 1), _s1_map),
        pl.BlockSpec((_NCP, _D), lambda g: (0, 0)),
        pl.BlockSpec((_BC, _D), _s2_map),
        pl.BlockSpec((_BX, _D), _s1_map),
        pl.BlockSpec((32, _NCP), lambda g: (0, 0)),
        pl.BlockSpec((32, _BC), _pdiag_map),
    ],
    out_specs=pl.BlockSpec(memory_space=pltpu.SMEM),
    out_shape=jax.ShapeDtypeStruct((4,), jnp.float32),
    compiler_params=pltpu.CompilerParams(
        dimension_semantics=("arbitrary",)),
)


_NW = 32
_BPW = _B // _NW


@functools.cache
def _sc_kernels():
    mesh = plsc.VectorSubcoreMesh(core_axis_name="c", subcore_axis_name="s")

    @functools.partial(
        pl.kernel,
        mesh=mesh,
        out_type=jax.ShapeDtypeStruct((_B, _D), jnp.float32),
        scratch_types=[
            pltpu.VMEM((_BPW,), jnp.int32),
            pltpu.VMEM((_BPW, _D), jnp.float32),
            pltpu.SemaphoreType.DMA,
        ],
        compiler_params=pltpu.CompilerParams(use_tc_tiling_on_sc=False),
    )
    def _sc_gather(labels_hbm, c_hbm, cb_hbm, idx_v, rows_v, sem):
        wid = lax.axis_index("s") * 2 + lax.axis_index("c")
        base = wid * _BPW
        pltpu.sync_copy(labels_hbm.at[pl.ds(base, _BPW)], idx_v)
        pltpu.async_copy(c_hbm.at[idx_v], rows_v, sem).wait()
        pltpu.sync_copy(rows_v, cb_hbm.at[pl.ds(base, _BPW)])

    @functools.partial(
        pl.kernel,
        mesh=mesh,
        out_type=jax.ShapeDtypeStruct((_NW, _NCP), jnp.float32),
        scratch_types=[
            pltpu.VMEM((_BPW,), jnp.int32),
            pltpu.VMEM((_NCP,), jnp.float32),
        ],
        compiler_params=pltpu.CompilerParams(needs_layout_passes=False),
    )
    def _sc_present(labels_hbm, present_hbm, idx_v, pbuf):
        wid = lax.axis_index("s") * 2 + lax.axis_index("c")
        base = wid * _BPW
        pltpu.sync_copy(labels_hbm.at[pl.ds(base, _BPW)], idx_v)

        def _zero(i, carry):
            pbuf[pl.ds(i * 16, 16)] = jnp.zeros((16,), jnp.float32)
            return carry

        lax.fori_loop(0, _NCP // 16, _zero, 0)
        ones16 = jnp.ones((16,), jnp.float32)
        for j in range(_BPW // 16):
            plsc.store_scatter(pbuf, [idx_v[pl.ds(j * 16, 16)]], ones16)
        pltpu.sync_copy(pbuf, present_hbm.at[wid])

    return _sc_present, _sc_gather


def _sc_part(labels, cpad):
    sc_present, sc_gather = _sc_kernels()
    return sc_present(labels), sc_gather(labels, cpad)


def kernel(x, labels, centers):
    c = centers.reshape(_NC, _D)
    cpad = jnp.pad(c, ((0, _NCP - _NC), (0, 0)))
    presentp, cb = _sc_part(labels, cpad)
    lab2 = labels.reshape(_B, 1)
    sums = _tc_call(x, lab2, cpad, cpad, cb, presentp, presentp)
    intraclass = sums[0] / (_B * _D * 2.0)
    triplet = sums[1] / (2.0 * _NC * _B)
    interclass = sums[2] / (_NC * _B * 2.0)
    return (intraclass, interclass, triplet)


# speedup vs baseline: 2.9190x; 1.0081x over previous
import functools

import jax
import jax.numpy as jnp
from jax import lax
from jax.experimental import pallas as pl
from jax.experimental.pallas import tpu as pltpu
from jax.experimental.pallas import tpu_sc as plsc

_B = 4096
_D = 32
_NSUB = 3
_NC = 3000
_NCP = 3072
_MARGIN = 1.0
_BX = 1024
_BC = 768
_NS1 = _B // _BX
_NS2 = _NCP // _BC
_NEG = -1e30


def _tc_body(x_ref, lab_ref, cfull_ref, crows_ref, cb_ref, pfull_ref,
             pdiag_ref, out_ref, caug_s, q_s):
    g = pl.program_id(0)
    ones_row = jnp.ones((1, _D), jnp.float32)

    @pl.when(g == 0)
    def _init():
        cfull = cfull_ref[...]
        c2_full = lax.dot_general(ones_row, cfull * cfull,
                                  (((1,), (1,)), ((), ())),
                                  preferred_element_type=jnp.float32)
        present = jnp.max(pfull_ref[...], axis=0, keepdims=True)
        colid = lax.broadcasted_iota(jnp.int32, (1, _NCP), 1)
        colcls = colid // _NSUB
        q_s[...] = jnp.where(present > 0.5, colcls, -1)
        negc2 = jnp.where(colid < _NC, -c2_full, _NEG)
        caug_s[...] = jnp.concatenate(
            [cfull, jnp.ones((_NCP, 1), jnp.float32),
             negc2.reshape(_NCP, 1)], axis=1)
        out_ref[0] = 0.0
        out_ref[1] = 0.0
        out_ref[2] = 0.0
        out_ref[3] = 0.0

    caug = caug_s[...]
    q = q_s[...]

    xb = x_ref[...]
    cbb = cb_ref[...]
    lab = lab_ref[...]
    diff = xb - cbb
    intra = jnp.sum(diff * diff, axis=1, keepdims=True)
    x2 = jnp.sum(xb * xb, axis=1, keepdims=True)
    b = _MARGIN + intra - x2
    xaug = jnp.concatenate([xb + xb, b, jnp.ones((_BX, 1), jnp.float32)],
                           axis=1)
    t = lax.dot_general(xaug, caug, (((1,), (1,)), ((), ())),
                        preferred_element_type=jnp.float32)
    lcls = lab // _NSUB
    r = jnp.where(q == lcls, 0.0, jnp.maximum(t, 0.0))
    out_ref[0] += jnp.sum(intra)
    out_ref[1] += jnp.sum(r)

    cr = crows_ref[...]
    c2col = jnp.sum(cr * cr, axis=1, keepdims=True)

    rowid = lax.broadcasted_iota(jnp.int32, (_BC, 1), 0) + g * _BC
    rowcls = rowid // _NSUB
    colid_d = lax.broadcasted_iota(jnp.int32, (1, _BC), 1) + g * _BC
    colcls_d = colid_d // _NSUB
    pd = jnp.max(pdiag_ref[...], axis=0, keepdims=True)
    crsq = cr * cr
    c2row_d = lax.dot_general(ones_row, crsq, (((1,), (1,)), ((), ())),
                              preferred_element_type=jnp.float32)
    ccd = lax.dot_general(cr, cr, (((1,), (1,)), ((), ())),
                          preferred_element_type=jnp.float32)
    dd = c2col + c2row_d - 2.0 * ccd

    samecls_d = rowcls == colcls_d
    eye = rowid == colid_d
    pdb = jnp.broadcast_to(pd, (_BC, _BC))
    pcol = jnp.max(jnp.where(eye, pdb, 0.0), axis=1,
                   keepdims=True)
    colm = jnp.max(jnp.where(samecls_d & (pcol > 0.5), dd, _NEG),
                   axis=0, keepdims=True)
    colmb = jnp.broadcast_to(colm, (_BC, _BC))
    dmax = jnp.max(jnp.where(samecls_d & (pdb > 0.5), colmb, _NEG),
                   axis=1, keepdims=True)
    cnt = jnp.sum(jnp.where(samecls_d, pdb, 0.0), axis=1,
                  keepdims=True)
    care = (cnt > 1.5).astype(jnp.float32)
    w = care * pcol

    b2 = jnp.where(w > 0.5, _MARGIN + dmax - c2col, _NEG)
    craug = jnp.concatenate([cr + cr, b2, jnp.ones((_BC, 1), jnp.float32)],
                            axis=1)
    t2 = lax.dot_general(craug, caug, (((1,), (1,)), ((), ())),
                         preferred_element_type=jnp.float32)
    r2 = jnp.maximum(t2, 0.0)
    t2d = 2.0 * ccd + b2 - c2row_d
    qd = jnp.where(pd > 0.5, colcls_d, -1)
    corr = jnp.where(qd == rowcls, jnp.maximum(t2d, 0.0), 0.0)
    out_ref[2] += jnp.sum(r2) - jnp.sum(corr)

    del out_ref


def _s1_map(g):
    return (g, 0)


def _s2_map(g):
    return (g, 0)


def _pdiag_map(g):
    return (0, g)


_tc_call = pl.pallas_call(
    _tc_body,
    grid=(_NS1,),
    in_specs=[
        pl.BlockSpec((_BX, _D), _s1_map),
        pl.BlockSpec((_BX, 1), _s1_map),
        pl.BlockSpec((_NCP, _D), lambda g: (0, 0)),
        pl.BlockSpec((_BC, _D), _s2_map),
        pl.BlockSpec((_BX, _D), _s1_map),
        pl.BlockSpec((32, _NCP), lambda g: (0, 0)),
        pl.BlockSpec((32, _BC), _pdiag_map),
    ],
    out_specs=pl.BlockSpec(memory_space=pltpu.SMEM),
    out_shape=jax.ShapeDtypeStruct((4,), jnp.float32),
    scratch_shapes=[
        pltpu.VMEM((_NCP, _D + 2), jnp.float32),
        pltpu.VMEM((1, _NCP), jnp.int32),
    ],
    compiler_params=pltpu.CompilerParams(
        dimension_semantics=("arbitrary",)),
)


_NW = 32
_BPW = _B // _NW


@functools.cache
def _sc_kernels():
    mesh = plsc.VectorSubcoreMesh(core_axis_name="c", subcore_axis_name="s")

    @functools.partial(
        pl.kernel,
        mesh=mesh,
        out_type=[
            jax.ShapeDtypeStruct((_NW, _NCP), jnp.float32),
            jax.ShapeDtypeStruct((_B, _D), jnp.float32),
        ],
        scratch_types=[
            pltpu.VMEM((_BPW,), jnp.int32),
            pltpu.VMEM((_BPW, _D), jnp.float32),
            pltpu.VMEM((_NCP,), jnp.float32),
            pltpu.SemaphoreType.DMA,
        ],
        compiler_params=pltpu.CompilerParams(
            use_tc_tiling_on_sc=False, needs_layout_passes=False),
    )
    def _sc_stage(labels_hbm, c_hbm, present_hbm, cb_hbm, idx_v, rows_v,
                  pbuf, sem):
        wid = lax.axis_index("s") * 2 + lax.axis_index("c")
        base = wid * _BPW
        pltpu.sync_copy(labels_hbm.at[pl.ds(base, _BPW)], idx_v)
        copy = pltpu.async_copy(c_hbm.at[idx_v], rows_v, sem)

        def _zero(i, carry):
            pbuf[pl.ds(i * 16, 16)] = jnp.zeros((16,), jnp.float32)
            return carry

        lax.fori_loop(0, _NCP // 16, _zero, 0)
        ones16 = jnp.ones((16,), jnp.float32)
        for j in range(_BPW // 16):
            plsc.store_scatter(pbuf, [idx_v[pl.ds(j * 16, 16)]], ones16)
        pltpu.sync_copy(pbuf, present_hbm.at[wid])

        copy.wait()
        pltpu.sync_copy(rows_v, cb_hbm.at[pl.ds(base, _BPW)])

    return _sc_stage


def _sc_part(labels, cpad):
    return _sc_kernels()(labels, cpad)


def kernel(x, labels, centers):
    c = centers.reshape(_NC, _D)
    cpad = jnp.pad(c, ((0, _NCP - _NC), (0, 0)))
    presentp, cb = _sc_part(labels, cpad)
    lab2 = labels.reshape(_B, 1)
    sums = _tc_call(x, lab2, cpad, cpad, cb, presentp, presentp)
    intraclass = sums[0] / (_B * _D * 2.0)
    triplet = sums[1] / (2.0 * _NC * _B)
    interclass = sums[2] / (_NC * _B * 2.0)
    return (intraclass, interclass, triplet)


# speedup vs baseline: 3.2380x; 1.1093x over previous
import functools

import jax
import jax.numpy as jnp
from jax import lax
from jax.experimental import pallas as pl
from jax.experimental.pallas import tpu as pltpu
from jax.experimental.pallas import tpu_sc as plsc

_B = 4096
_D = 32
_DW = 128
_LLANE = 32
_NSUB = 3
_NC = 3000
_NCP = 3072
_MARGIN = 1.0
_BX = 1024
_BC = 768
_BD = 384
_NS1 = _B // _BX
_NEG = -1e30


def _tc_body(x_ref, cfull_ref, crows_ref, cb_ref, pfull_ref,
             pdiag_ref, out_ref, caug_s, q_s):
    g = pl.program_id(0)
    ones_row = jnp.ones((1, _D), jnp.float32)

    @pl.when(g == 0)
    def _init():
        cfull = cfull_ref[...]
        c2_full = lax.dot_general(ones_row, cfull * cfull,
                                  (((1,), (1,)), ((), ())),
                                  preferred_element_type=jnp.float32)
        present = jnp.max(pfull_ref[...], axis=0, keepdims=True)
        colid = lax.broadcasted_iota(jnp.int32, (1, _NCP), 1)
        colcls = colid // _NSUB
        q_s[...] = jnp.where(present > 0.5, colcls, -1)
        negc2 = jnp.where(colid < _NC, -c2_full, _NEG)
        caug_s[...] = jnp.concatenate(
            [cfull, jnp.ones((_NCP, 1), jnp.float32),
             negc2.reshape(_NCP, 1)], axis=1)
        out_ref[0] = 0.0
        out_ref[1] = 0.0
        out_ref[2] = 0.0
        out_ref[3] = 0.0

    caug = caug_s[...]
    q = q_s[...]

    xb = x_ref[...]
    cbw = cb_ref[...]
    cbb = cbw[:, :_D]
    lab = cbw[:, _LLANE:_LLANE + 1].astype(jnp.int32)
    diff = xb - cbb
    intra = jnp.sum(diff * diff, axis=1, keepdims=True)
    x2 = jnp.sum(xb * xb, axis=1, keepdims=True)
    b = _MARGIN + intra - x2
    xaug = jnp.concatenate([xb + xb, b, jnp.ones((_BX, 1), jnp.float32)],
                           axis=1)
    t = lax.dot_general(xaug, caug, (((1,), (1,)), ((), ())),
                        preferred_element_type=jnp.float32)
    lcls = lab // _NSUB
    r = jnp.where(q == lcls, 0.0, jnp.maximum(t, 0.0))
    out_ref[0] += jnp.sum(intra)
    out_ref[1] += jnp.sum(r)

    cr_full = crows_ref[...]
    s2 = jnp.float32(0.0)
    for h in range(_BC // _BD):
        cr = cr_full[h * _BD:(h + 1) * _BD]
        c2col = jnp.sum(cr * cr, axis=1, keepdims=True)
        base = g * _BC + h * _BD
        rowid = lax.broadcasted_iota(jnp.int32, (_BD, 1), 0) + base
        rowcls = rowid // _NSUB
        colid_d = lax.broadcasted_iota(jnp.int32, (1, _BD), 1) + base
        colcls_d = colid_d // _NSUB
        pd = jnp.max(pdiag_ref[:, h * _BD:(h + 1) * _BD], axis=0,
                     keepdims=True)
        crsq = cr * cr
        c2row_d = lax.dot_general(ones_row, crsq, (((1,), (1,)), ((), ())),
                                  preferred_element_type=jnp.float32)
        ccd = lax.dot_general(cr, cr, (((1,), (1,)), ((), ())),
                              preferred_element_type=jnp.float32)
        dd = c2col + c2row_d - 2.0 * ccd

        samecls_d = rowcls == colcls_d
        eye = rowid == colid_d
        pdb = jnp.broadcast_to(pd, (_BD, _BD))
        pcol = jnp.max(jnp.where(eye, pdb, 0.0), axis=1,
                       keepdims=True)
        colm = jnp.max(jnp.where(samecls_d & (pcol > 0.5), dd, _NEG),
                       axis=0, keepdims=True)
        colmb = jnp.broadcast_to(colm, (_BD, _BD))
        dmax = jnp.max(jnp.where(samecls_d & (pdb > 0.5), colmb, _NEG),
                       axis=1, keepdims=True)
        cnt = jnp.sum(jnp.where(samecls_d, pdb, 0.0), axis=1,
                      keepdims=True)
        care = (cnt > 1.5).astype(jnp.float32)
        w = care * pcol

        b2 = jnp.where(w > 0.5, _MARGIN + dmax - c2col, _NEG)
        craug = jnp.concatenate(
            [cr + cr, b2, jnp.ones((_BD, 1), jnp.float32)], axis=1)
        t2 = lax.dot_general(craug, caug, (((1,), (1,)), ((), ())),
                             preferred_element_type=jnp.float32)
        r2 = jnp.maximum(t2, 0.0)
        t2d = 2.0 * ccd + b2 - c2row_d
        qd = jnp.where(pd > 0.5, colcls_d, -1)
        corr = jnp.where(qd == rowcls, jnp.maximum(t2d, 0.0), 0.0)
        s2 = s2 + (jnp.sum(r2) - jnp.sum(corr))
    out_ref[2] += s2


def _s1_map(g):
    return (g, 0)


def _s2_map(g):
    return (g, 0)


def _pdiag_map(g):
    return (0, g)


_TC_KW = dict(
    grid=(_NS1,),
    in_specs=[
        pl.BlockSpec((_BX, _D), _s1_map),
        pl.BlockSpec((_NCP, _D), lambda g: (0, 0)),
        pl.BlockSpec((_BC, _D), _s2_map),
        pl.BlockSpec((_BX, _DW), _s1_map),
        pl.BlockSpec((32, _NCP), lambda g: (0, 0)),
        pl.BlockSpec((32, _BC), _pdiag_map),
    ],
    out_specs=pl.BlockSpec(memory_space=pltpu.SMEM),
    out_shape=jax.ShapeDtypeStruct((4,), jnp.float32),
    scratch_shapes=[
        pltpu.VMEM((_NCP, _D + 2), jnp.float32),
        pltpu.VMEM((1, _NCP), jnp.int32),
    ],
    compiler_params=pltpu.CompilerParams(
        dimension_semantics=("arbitrary",)),
)

_tc_call = pl.pallas_call(_tc_body, **_TC_KW)


_NW = 32
_BPW = _B // _NW


@functools.cache
def _sc_kernels():
    mesh = plsc.VectorSubcoreMesh(core_axis_name="c", subcore_axis_name="s")

    @functools.partial(
        pl.kernel,
        mesh=mesh,
        out_type=[
            jax.ShapeDtypeStruct((_NW, _NCP), jnp.float32),
            jax.ShapeDtypeStruct((_B, _DW), jnp.float32),
        ],
        scratch_types=[
            pltpu.VMEM((_BPW,), jnp.int32),
            pltpu.VMEM((_BPW, _DW), jnp.float32),
            pltpu.VMEM((_NCP,), jnp.float32),
            pltpu.SemaphoreType.DMA,
        ],
        compiler_params=pltpu.CompilerParams(needs_layout_passes=False),
    )
    def _sc_stage(labels_hbm, c_hbm, present_hbm, cb_hbm, idx_v, rows_v,
                  pbuf, sem):
        wid = lax.axis_index("s") * 2 + lax.axis_index("c")
        base = wid * _BPW
        pltpu.sync_copy(labels_hbm.at[pl.ds(base, _BPW)], idx_v)
        copy = pltpu.async_copy(c_hbm.at[idx_v], rows_v, sem)

        def _zero(i, carry):
            pbuf[pl.ds(i * 16, 16)] = jnp.zeros((16,), jnp.float32)
            return carry

        lax.fori_loop(0, _NCP // 16, _zero, 0)
        ones16 = jnp.ones((16,), jnp.float32)
        for j in range(_BPW // 16):
            plsc.store_scatter(pbuf, [idx_v[pl.ds(j * 16, 16)]], ones16)
        pltpu.sync_copy(pbuf, present_hbm.at[wid])

        copy.wait()
        lane = jnp.full((16,), _LLANE, jnp.int32)
        for j in range(_BPW // 16):
            ridx = lax.broadcasted_iota(jnp.int32, (16,), 0) + j * 16
            vals = idx_v[pl.ds(j * 16, 16)].astype(jnp.float32)
            plsc.store_scatter(rows_v, [ridx, lane], vals)
        pltpu.sync_copy(rows_v, cb_hbm.at[pl.ds(base, _BPW)])

    return _sc_stage


def _sc_part(labels, cwide):
    return _sc_kernels()(labels, cwide)


def kernel(x, labels, centers):
    c = centers.reshape(_NC, _D)
    cpad = jnp.pad(c, ((0, _NCP - _NC), (0, 0)))
    cwide = jnp.pad(c, ((0, _NCP - _NC), (0, _DW - _D)))
    presentp, cbw = _sc_part(labels, cwide)
    sums = _tc_call(x, cpad, cpad, cbw, presentp, presentp)
    intraclass = sums[0] / (_B * _D * 2.0)
    triplet = sums[1] / (2.0 * _NC * _B)
    interclass = sums[2] / (_NC * _B * 2.0)
    return (intraclass, interclass, triplet)
